# baseline (device time: 218710 ns/iter reference)
import jax
import jax.numpy as jnp
from jax import lax
from jax.experimental import pallas as pl
from jax.experimental.pallas import tpu as pltpu

N_DEV = 4
M_PER = 2048
K = 8192
N_PER = 1024
K_BLK = 512
K_TILES = K // K_BLK
N_TILES = N_DEV * K_TILES
PERM = (2, 1, 3, 0)


def _silu(y):
    return y * (1.0 / (1.0 + jnp.exp(-y)))


def kernel(x, w_mat):
    my_i = lax.axis_index("i")
    targets = jnp.mod(my_i + jnp.array(PERM, jnp.int32), N_DEV)

    def body(tgt_ref, x_ref, w_ref, out_ref,
             acc_ref, x_bf, w_bf, send_buf, recv_buf,
             send_sems, recv_sems, copy_sems):
        s = pl.program_id(0)
        my = lax.axis_index("i")

        @pl.when(s == 0)
        def _entry_barrier():
            bsem = pltpu.get_barrier_semaphore()
            for off in (1, 2, 3):
                pl.semaphore_signal(
                    bsem, inc=1,
                    device_id=(lax.rem(my + off, N_DEV),),
                    device_id_type=pl.DeviceIdType.MESH,
                )
            pl.semaphore_wait(bsem, N_DEV - 1)

        @pl.when(lax.rem(s, K_TILES) == 1)
        def _init_acc():
            acc_ref[...] = jnp.zeros_like(acc_ref)

        d = lax.rem(s, 2)
        x_bf[d] = x_ref[...].astype(jnp.bfloat16)
        w_bf[d] = w_ref[...].astype(jnp.bfloat16)
        prev = 1 - d
        acc_ref[...] += jnp.dot(
            x_bf[prev], w_bf[prev], preferred_element_type=jnp.float32
        )

        def out_rows(src):
            return out_ref.at[pl.ds(src * M_PER, M_PER), :]

        for p in range(3):
            @pl.when(s == (p + 1) * K_TILES)
            def _send(p=p):
                send_buf[p] = _silu(acc_ref[...]).astype(jnp.bfloat16)
                rdma = pltpu.make_async_remote_copy(
                    src_ref=send_buf.at[p],
                    dst_ref=recv_buf.at[p],
                    send_sem=send_sems.at[p],
                    recv_sem=recv_sems.at[p],
                    device_id=(lax.rem(my + PERM[p], N_DEV),),
                    device_id_type=pl.DeviceIdType.MESH,
                )
                rdma.start()

        @pl.when(s == N_TILES)
        def _finish():
            send_buf[3] = _silu(acc_ref[...]).astype(jnp.bfloat16)
            pltpu.make_async_copy(
                send_buf.at[3], out_rows(my), copy_sems.at[3]
            ).start()

            for p in range(3):
                recv = pltpu.make_async_remote_copy(
                    src_ref=send_buf.at[p],
                    dst_ref=recv_buf.at[p],
                    send_sem=send_sems.at[p],
                    recv_sem=recv_sems.at[p],
                    device_id=(my,),
                    device_id_type=pl.DeviceIdType.MESH,
                )
                recv.wait_recv()
                src = lax.rem(my - PERM[p] + N_DEV, N_DEV)
                pltpu.make_async_copy(
                    recv_buf.at[p], out_rows(src), copy_sems.at[p]
                ).start()

            for c in range(4):
                pltpu.make_async_copy(
                    send_buf.at[3], out_rows(my), copy_sems.at[c]
                ).wait()
            for p in range(3):
                send = pltpu.make_async_remote_copy(
                    src_ref=send_buf.at[p],
                    dst_ref=recv_buf.at[p],
                    send_sem=send_sems.at[p],
                    recv_sem=recv_sems.at[p],
                    device_id=(my,),
                    device_id_type=pl.DeviceIdType.MESH,
                )
                send.wait_send()

    def x_map(s, tgt):
        return (0, lax.rem(jnp.minimum(s, N_TILES - 1), K_TILES))

    def w_map(s, tgt):
        ss = jnp.minimum(s, N_TILES - 1)
        return (lax.rem(ss, K_TILES), tgt[ss // K_TILES])

    grid_spec = pltpu.PrefetchScalarGridSpec(
        num_scalar_prefetch=1,
        grid=(N_TILES + 1,),
        in_specs=[
            pl.BlockSpec((M_PER, K_BLK), x_map),
            pl.BlockSpec((K_BLK, N_PER), w_map),
        ],
        out_specs=pl.BlockSpec(memory_space=pl.ANY),
        scratch_shapes=[
            pltpu.VMEM((M_PER, N_PER), jnp.float32),
            pltpu.VMEM((2, M_PER, K_BLK), jnp.bfloat16),
            pltpu.VMEM((2, K_BLK, N_PER), jnp.bfloat16),
            pltpu.VMEM((4, M_PER, N_PER), jnp.bfloat16),
            pltpu.VMEM((3, M_PER, N_PER), jnp.bfloat16),
            pltpu.SemaphoreType.DMA((3,)),
            pltpu.SemaphoreType.DMA((3,)),
            pltpu.SemaphoreType.DMA((4,)),
        ],
    )

    return pl.pallas_call(
        body,
        grid_spec=grid_spec,
        out_shape=jax.ShapeDtypeStruct((N_DEV * M_PER, N_PER), jnp.bfloat16),
        compiler_params=pltpu.CompilerParams(
            collective_id=0,
            dimension_semantics=("arbitrary",),
            vmem_limit_bytes=60 * 1024 * 1024,
        ),
    )(targets, x, w_mat)


# device time: 211801 ns/iter; 1.0326x vs baseline; 1.0326x over previous
import jax
import jax.numpy as jnp
from jax import lax
from jax.experimental import pallas as pl
from jax.experimental.pallas import tpu as pltpu

DIAG_NO_CAST = True

N_DEV = 4
M_PER = 2048
K = 8192
N_PER = 1024
K_BLK = 512
K_TILES = K // K_BLK
N_TILES = N_DEV * K_TILES
PERM = (2, 1, 3, 0)


def _silu(y):
    return y * (1.0 / (1.0 + jnp.exp(-y)))


def kernel(x, w_mat):
    my_i = lax.axis_index("i")
    targets = jnp.mod(my_i + jnp.array(PERM, jnp.int32), N_DEV)

    def body(tgt_ref, x_ref, w_ref, out_ref,
             acc_ref, x_bf, w_bf, send_buf, recv_buf,
             send_sems, recv_sems, copy_sems):
        s = pl.program_id(0)
        my = lax.axis_index("i")

        @pl.when(s == 0)
        def _entry_barrier():
            bsem = pltpu.get_barrier_semaphore()
            for off in (1, 2, 3):
                pl.semaphore_signal(
                    bsem, inc=1,
                    device_id=(lax.rem(my + off, N_DEV),),
                    device_id_type=pl.DeviceIdType.MESH,
                )
            pl.semaphore_wait(bsem, N_DEV - 1)

        @pl.when(lax.rem(s, K_TILES) == 1)
        def _init_acc():
            acc_ref[...] = jnp.zeros_like(acc_ref)

        d = lax.rem(s, 2)
        if not DIAG_NO_CAST:
            x_bf[d] = x_ref[...].astype(jnp.bfloat16)
            w_bf[d] = w_ref[...].astype(jnp.bfloat16)
        prev = 1 - d
        acc_ref[...] += jnp.dot(
            x_bf[prev], w_bf[prev], preferred_element_type=jnp.float32
        )

        def out_rows(src):
            return out_ref.at[pl.ds(src * M_PER, M_PER), :]

        for p in range(3):
            @pl.when(s == (p + 1) * K_TILES)
            def _send(p=p):
                send_buf[p] = _silu(acc_ref[...]).astype(jnp.bfloat16)
                rdma = pltpu.make_async_remote_copy(
                    src_ref=send_buf.at[p],
                    dst_ref=recv_buf.at[p],
                    send_sem=send_sems.at[p],
                    recv_sem=recv_sems.at[p],
                    device_id=(lax.rem(my + PERM[p], N_DEV),),
                    device_id_type=pl.DeviceIdType.MESH,
                )
                rdma.start()

        @pl.when(s == N_TILES)
        def _finish():
            send_buf[3] = _silu(acc_ref[...]).astype(jnp.bfloat16)
            pltpu.make_async_copy(
                send_buf.at[3], out_rows(my), copy_sems.at[3]
            ).start()

            for p in range(3):
                recv = pltpu.make_async_remote_copy(
                    src_ref=send_buf.at[p],
                    dst_ref=recv_buf.at[p],
                    send_sem=send_sems.at[p],
                    recv_sem=recv_sems.at[p],
                    device_id=(my,),
                    device_id_type=pl.DeviceIdType.MESH,
                )
                recv.wait_recv()
                src = lax.rem(my - PERM[p] + N_DEV, N_DEV)
                pltpu.make_async_copy(
                    recv_buf.at[p], out_rows(src), copy_sems.at[p]
                ).start()

            for c in range(4):
                pltpu.make_async_copy(
                    send_buf.at[3], out_rows(my), copy_sems.at[c]
                ).wait()
            for p in range(3):
                send = pltpu.make_async_remote_copy(
                    src_ref=send_buf.at[p],
                    dst_ref=recv_buf.at[p],
                    send_sem=send_sems.at[p],
                    recv_sem=recv_sems.at[p],
                    device_id=(my,),
                    device_id_type=pl.DeviceIdType.MESH,
                )
                send.wait_send()

    def x_map(s, tgt):
        return (0, lax.rem(jnp.minimum(s, N_TILES - 1), K_TILES))

    def w_map(s, tgt):
        ss = jnp.minimum(s, N_TILES - 1)
        return (lax.rem(ss, K_TILES), tgt[ss // K_TILES])

    grid_spec = pltpu.PrefetchScalarGridSpec(
        num_scalar_prefetch=1,
        grid=(N_TILES + 1,),
        in_specs=[
            pl.BlockSpec((M_PER, K_BLK), x_map),
            pl.BlockSpec((K_BLK, N_PER), w_map),
        ],
        out_specs=pl.BlockSpec(memory_space=pl.ANY),
        scratch_shapes=[
            pltpu.VMEM((M_PER, N_PER), jnp.float32),
            pltpu.VMEM((2, M_PER, K_BLK), jnp.bfloat16),
            pltpu.VMEM((2, K_BLK, N_PER), jnp.bfloat16),
            pltpu.VMEM((4, M_PER, N_PER), jnp.bfloat16),
            pltpu.VMEM((3, M_PER, N_PER), jnp.bfloat16),
            pltpu.SemaphoreType.DMA((3,)),
            pltpu.SemaphoreType.DMA((3,)),
            pltpu.SemaphoreType.DMA((4,)),
        ],
    )

    return pl.pallas_call(
        body,
        grid_spec=grid_spec,
        out_shape=jax.ShapeDtypeStruct((N_DEV * M_PER, N_PER), jnp.bfloat16),
        compiler_params=pltpu.CompilerParams(
            collective_id=0,
            dimension_semantics=("arbitrary",),
            vmem_limit_bytes=60 * 1024 * 1024,
        ),
    )(targets, x, w_mat)


# device time: 196159 ns/iter; 1.1150x vs baseline; 1.0797x over previous
import jax
import jax.numpy as jnp
from jax import lax
from jax.experimental import pallas as pl
from jax.experimental.pallas import tpu as pltpu

N_DEV = 4
M_PER = 2048
K = 8192
N_PER = 1024
K_BLK = 1024
K_TILES = K // K_BLK
N_TILES = N_DEV * K_TILES
PERM = (2, 1, 3, 0)
SEND_SLOT = (0, 1, 0)


def _silu(y):
    return y * (1.0 / (1.0 + jnp.exp(-y)))


def kernel(x, w_mat):
    my_i = lax.axis_index("i")
    targets = jnp.mod(my_i + jnp.array(PERM, jnp.int32), N_DEV)

    def body(tgt_ref, x_ref, w_ref, out_ref,
             acc_ref, send_buf, recv_buf,
             send_sems, recv_sems, copy_sems):
        s = pl.program_id(0)
        my = lax.axis_index("i")

        def send_descriptor(slot, p):
            return pltpu.make_async_remote_copy(
                src_ref=send_buf.at[slot],
                dst_ref=recv_buf.at[p],
                send_sem=send_sems.at[slot],
                recv_sem=recv_sems.at[p],
                device_id=(lax.rem(my + PERM[p], N_DEV),),
                device_id_type=pl.DeviceIdType.MESH,
            )

        @pl.when(s == 0)
        def _entry_barrier():
            bsem = pltpu.get_barrier_semaphore()
            for off in (1, 2, 3):
                pl.semaphore_signal(
                    bsem, inc=1,
                    device_id=(lax.rem(my + off, N_DEV),),
                    device_id_type=pl.DeviceIdType.MESH,
                )
            pl.semaphore_wait(bsem, N_DEV - 1)

        @pl.when(lax.rem(s, K_TILES) == 0)
        def _init_acc():
            acc_ref[...] = jnp.zeros_like(acc_ref)

        acc_ref[...] += jnp.dot(
            x_ref[...].astype(jnp.bfloat16),
            w_ref[...].astype(jnp.bfloat16),
            preferred_element_type=jnp.float32,
        )

        def out_rows(src):
            return out_ref.at[pl.ds(src * M_PER, M_PER), :]

        for p in range(3):
            @pl.when(s == (p + 1) * K_TILES - 1)
            def _send(p=p):
                slot = SEND_SLOT[p]
                if p == 2:
                    send_descriptor(0, 0).wait_send()
                send_buf[slot] = _silu(acc_ref[...]).astype(jnp.bfloat16)
                send_descriptor(slot, p).start()

        @pl.when(s == N_TILES - 1)
        def _finish():
            send_descriptor(1, 1).wait_send()
            send_buf[1] = _silu(acc_ref[...]).astype(jnp.bfloat16)
            pltpu.make_async_copy(
                send_buf.at[1], out_rows(my), copy_sems.at[3]
            ).start()

            for p in range(3):
                send_descriptor(SEND_SLOT[p], p).wait_recv()
                src = lax.rem(my - PERM[p] + N_DEV, N_DEV)
                pltpu.make_async_copy(
                    recv_buf.at[p], out_rows(src), copy_sems.at[p]
                ).start()

            for c in range(4):
                pltpu.make_async_copy(
                    send_buf.at[1], out_rows(my), copy_sems.at[c]
                ).wait()
            send_descriptor(0, 2).wait_send()

    def x_map(s, tgt):
        return (0, lax.rem(s, K_TILES))

    def w_map(s, tgt):
        return (lax.rem(s, K_TILES), tgt[s // K_TILES])

    grid_spec = pltpu.PrefetchScalarGridSpec(
        num_scalar_prefetch=1,
        grid=(N_TILES,),
        in_specs=[
            pl.BlockSpec((M_PER, K_BLK), x_map),
            pl.BlockSpec((K_BLK, N_PER), w_map),
        ],
        out_specs=pl.BlockSpec(memory_space=pl.ANY),
        scratch_shapes=[
            pltpu.VMEM((M_PER, N_PER), jnp.float32),
            pltpu.VMEM((2, M_PER, N_PER), jnp.bfloat16),
            pltpu.VMEM((3, M_PER, N_PER), jnp.bfloat16),
            pltpu.SemaphoreType.DMA((2,)),
            pltpu.SemaphoreType.DMA((3,)),
            pltpu.SemaphoreType.DMA((4,)),
        ],
    )

    return pl.pallas_call(
        body,
        grid_spec=grid_spec,
        out_shape=jax.ShapeDtypeStruct((N_DEV * M_PER, N_PER), jnp.bfloat16),
        compiler_params=pltpu.CompilerParams(
            collective_id=0,
            dimension_semantics=("arbitrary",),
            vmem_limit_bytes=60 * 1024 * 1024,
        ),
    )(targets, x, w_mat)
